# baseline (speedup 1.0000x reference)
import jax
import jax.numpy as jnp
from jax.experimental import pallas as pl
from jax.experimental.pallas import tpu as pltpu

B, D, K = 4096, 2048, 8
BM = 512
BC = 128
NBLK = B // BM
NCHUNK = BM // BC


def _encode_block(idx_ref, mask_ref, x_ref, w_hbm, b_ref, out_ref,
                  wtile, wbtile, w_sem):
    i = pl.program_id(0)
    idx = idx_ref[0]

    @pl.when(i == 0)
    def _():
        cp = pltpu.make_async_copy(w_hbm, wtile, w_sem)
        cp.start()
        cp.wait()
        wbtile[...] = wtile[...].astype(jnp.bfloat16)

    onehot = (jax.lax.broadcasted_iota(jnp.int32, (1, K), 1) == idx)
    sel = jnp.sum(mask_ref[...] * onehot.astype(jnp.float32), axis=1,
                  keepdims=True)
    keep = sel > 0.5
    wb = wbtile[...]
    bias = b_ref[...]
    for c in range(NCHUNK):
        lo, hi = c * BC, (c + 1) * BC
        xb = x_ref[lo:hi, :].astype(jnp.bfloat16)
        acc = jax.lax.dot_general(
            xb, wb, (((1,), (1,)), ((), ())),
            preferred_element_type=jnp.float32)
        out_ref[lo:hi, :] = jnp.where(keep[lo:hi, :], acc + bias, 0.0)


def kernel(input_data, selection_mask, W, bvec, modality_idx):
    idx = jnp.atleast_1d(jnp.asarray(modality_idx, dtype=jnp.int32))
    grid_spec = pltpu.PrefetchScalarGridSpec(
        num_scalar_prefetch=1,
        grid=(NBLK,),
        in_specs=[
            pl.BlockSpec((BM, K), lambda i, *_: (i, 0)),
            pl.BlockSpec((BM, D), lambda i, *_: (i, 0)),
            pl.BlockSpec(memory_space=pl.ANY),
            pl.BlockSpec((1, D), lambda i, *_: (0, 0)),
        ],
        out_specs=pl.BlockSpec((BM, D), lambda i, *_: (i, 0)),
        scratch_shapes=[
            pltpu.VMEM((D, D), jnp.float32),
            pltpu.VMEM((D, D), jnp.bfloat16),
            pltpu.SemaphoreType.DMA,
        ],
    )
    return pl.pallas_call(
        _encode_block,
        grid_spec=grid_spec,
        out_shape=jax.ShapeDtypeStruct((B, D), jnp.float32),
    )(idx, selection_mask, input_data, W, bvec.reshape(1, D))


# speedup vs baseline: 1.8500x; 1.8500x over previous
import jax
import jax.numpy as jnp
from jax.experimental import pallas as pl
from jax.experimental.pallas import tpu as pltpu

B, D, K = 4096, 2048, 8
BM = 1024
NBLK = B // BM
WCH = 512


def _encode_block(idx_ref, mask_ref, x_ref, w_hbm, b_ref, out_ref,
                  wstage, wbtile, w_sem):
    i = pl.program_id(0)
    idx = idx_ref[0]

    @pl.when(i == 0)
    def _():
        for c in range(D // WCH):
            cp = pltpu.make_async_copy(
                w_hbm.at[pl.ds(c * WCH, WCH), :], wstage, w_sem)
            cp.start()
            cp.wait()
            wbtile[c * WCH:(c + 1) * WCH, :] = wstage[...].astype(
                jnp.bfloat16)

    onehot = (jax.lax.broadcasted_iota(jnp.int32, (1, K), 1) == idx)
    sel = jnp.sum(mask_ref[...] * onehot.astype(jnp.float32), axis=1,
                  keepdims=True)
    keep = sel > 0.5
    xb = x_ref[...].astype(jnp.bfloat16)
    acc = jax.lax.dot_general(
        xb, wbtile[...], (((1,), (1,)), ((), ())),
        preferred_element_type=jnp.float32)
    out_ref[...] = jnp.where(keep, acc + b_ref[...], 0.0)


def kernel(input_data, selection_mask, W, bvec, modality_idx):
    idx = jnp.atleast_1d(jnp.asarray(modality_idx, dtype=jnp.int32))
    grid_spec = pltpu.PrefetchScalarGridSpec(
        num_scalar_prefetch=1,
        grid=(NBLK,),
        in_specs=[
            pl.BlockSpec((BM, K), lambda i, *_: (i, 0)),
            pl.BlockSpec((BM, D), lambda i, *_: (i, 0)),
            pl.BlockSpec(memory_space=pl.ANY),
            pl.BlockSpec((1, D), lambda i, *_: (0, 0)),
        ],
        out_specs=pl.BlockSpec((BM, D), lambda i, *_: (i, 0)),
        scratch_shapes=[
            pltpu.VMEM((WCH, D), jnp.float32),
            pltpu.VMEM((D, D), jnp.bfloat16),
            pltpu.SemaphoreType.DMA,
        ],
    )
    return pl.pallas_call(
        _encode_block,
        grid_spec=grid_spec,
        out_shape=jax.ShapeDtypeStruct((B, D), jnp.float32),
    )(idx, selection_mask, input_data, W, bvec.reshape(1, D))


# speedup vs baseline: 2.1200x; 1.1459x over previous
import jax
import jax.numpy as jnp
from jax.experimental import pallas as pl
from jax.experimental.pallas import tpu as pltpu

B, D, K = 4096, 2048, 8
BM = 512


def _encode_block(idx_ref, mask_ref, x_ref, w_ref, b_ref, out_ref):
    idx = idx_ref[0]
    onehot = (jax.lax.broadcasted_iota(jnp.int32, (1, K), 1) == idx)
    sel = jnp.sum(mask_ref[...] * onehot.astype(jnp.float32), axis=1,
                  keepdims=True)
    keep = sel > 0.5
    xb = x_ref[...].astype(jnp.bfloat16)
    wb = w_ref[...].astype(jnp.bfloat16)
    acc = jax.lax.dot_general(
        xb, wb, (((1,), (1,)), ((), ())),
        preferred_element_type=jnp.float32)
    acc = acc + b_ref[...]
    out_ref[...] = jnp.where(keep, acc, 0.0)


def kernel(input_data, selection_mask, W, bvec, modality_idx):
    idx = jnp.atleast_1d(jnp.asarray(modality_idx, dtype=jnp.int32))
    grid_spec = pltpu.PrefetchScalarGridSpec(
        num_scalar_prefetch=1,
        grid=(B // BM,),
        in_specs=[
            pl.BlockSpec((BM, K), lambda i, *_: (i, 0)),
            pl.BlockSpec((BM, D), lambda i, *_: (i, 0)),
            pl.BlockSpec((D, D), lambda i, *_: (0, 0)),
            pl.BlockSpec((1, D), lambda i, *_: (0, 0)),
        ],
        out_specs=pl.BlockSpec((BM, D), lambda i, *_: (i, 0)),
    )
    return pl.pallas_call(
        _encode_block,
        grid_spec=grid_spec,
        out_shape=jax.ShapeDtypeStruct((B, D), jnp.float32),
        compiler_params=pltpu.CompilerParams(
            dimension_semantics=("parallel",),
            fuse_transposed_lhs_in_matmul=True),
    )(idx, selection_mask, input_data, W, bvec.reshape(1, D))


# speedup vs baseline: 2.1257x; 1.0027x over previous
import jax
import jax.numpy as jnp
from jax.experimental import pallas as pl
from jax.experimental.pallas import tpu as pltpu

B, D, K = 4096, 2048, 8
BM = 512


def _encode_block(idx_ref, mask_ref, x_ref, w_ref, b_ref, out_ref):
    idx = idx_ref[0]
    onehot = (jax.lax.broadcasted_iota(jnp.int32, (1, K), 1) == idx)
    sel = jnp.sum(mask_ref[...] * onehot.astype(jnp.float32), axis=1,
                  keepdims=True)
    keep = sel > 0.5
    xb = x_ref[...].astype(jnp.bfloat16)
    wb = w_ref[...].astype(jnp.bfloat16)
    acc = jax.lax.dot_general(
        xb, wb, (((1,), (1,)), ((), ())),
        preferred_element_type=jnp.float32)
    acc = acc + b_ref[...][None, :]
    out_ref[...] = jnp.where(keep, acc, 0.0)


def kernel(input_data, selection_mask, W, bvec, modality_idx):
    idx = jnp.atleast_1d(jnp.asarray(modality_idx, dtype=jnp.int32))
    grid_spec = pltpu.PrefetchScalarGridSpec(
        num_scalar_prefetch=1,
        grid=(B // BM,),
        in_specs=[
            pl.BlockSpec((BM, K), lambda i, *_: (i, 0)),
            pl.BlockSpec((BM, D), lambda i, *_: (i, 0)),
            pl.BlockSpec((D, D), lambda i, *_: (0, 0)),
            pl.BlockSpec((D,), lambda i, *_: (0,)),
        ],
        out_specs=pl.BlockSpec((BM, D), lambda i, *_: (i, 0)),
    )
    return pl.pallas_call(
        _encode_block,
        grid_spec=grid_spec,
        out_shape=jax.ShapeDtypeStruct((B, D), jnp.float32),
        compiler_params=pltpu.CompilerParams(
            dimension_semantics=("parallel",),
            fuse_transposed_lhs_in_matmul=True),
    )(idx, selection_mask, input_data, W, bvec)
